# baseline (speedup 1.0000x reference)
import functools

import jax
import jax.numpy as jnp
from jax import lax
from jax.experimental import pallas as pl
from jax.experimental.pallas import tpu as pltpu
from jax.experimental.pallas import tpu_sc as plsc

_LANES = 16
_BB = 128
_TW = 128


@functools.lru_cache(maxsize=None)
def _make_transpose(vocab, d_model):
    info = plsc.get_sparse_core_info()
    nc, ns = info.num_cores, info.num_subcores
    nw = nc * ns
    cvecs = d_model // _LANES
    full_blocks = vocab // _TW
    blocks = full_blocks // nw
    extra = full_blocks - blocks * nw
    tail = vocab - full_blocks * _TW

    mesh = plsc.VectorSubcoreMesh(core_axis_name="c", subcore_axis_name="s")

    scratch = [pltpu.VMEM((d_model, _TW), jnp.float32) for _ in range(2)]
    scratch += [pltpu.VMEM((_TW // 2, 2 * d_model), jnp.float32) for _ in range(2)]
    scratch += [pltpu.SemaphoreType.DMA for _ in range(4)]

    @functools.partial(
        pl.kernel,
        mesh=mesh,
        out_type=jax.ShapeDtypeStruct(
            ((vocab + 1) // 2, 2 * d_model), jnp.float32),
        scratch_types=scratch,
        compiler_params=pltpu.CompilerParams(
            use_tc_tiling_on_sc=True, needs_layout_passes=False),
    )
    def tr(tblt_hbm, tail_hbm, out_hbm, slab0, slab1, bout0, bout1,
           g0, g1, s0, s1):
        slabs = (slab0, slab1)
        bouts = (bout0, bout1)
        gsems = (g0, g1)
        ssems = (s0, s1)

        wid = lax.axis_index("s") * nc + lax.axis_index("c")
        base = wid * blocks * _TW

        iotas = [lax.iota(jnp.int32, _LANES) + c * _LANES for c in range(cvecs)]

        def fire_read(t0, bb, w=_TW):
            t0 = pl.multiple_of(t0, _TW)
            pltpu.async_copy(
                tblt_hbm.at[:, pl.ds(t0, w)], slabs[bb].at[:, pl.ds(0, w)],
                gsems[bb])

        def wait_read(bb, w=_TW):
            pltpu.make_async_copy(
                tblt_hbm.at[:, pl.ds(0, w)], slabs[bb].at[:, pl.ds(0, w)],
                gsems[bb]).wait()

        def fire_write(t0, bb, w=_TW):
            r0 = pl.multiple_of(t0 // 2, _TW // 2)
            pltpu.async_copy(
                bouts[bb].at[pl.ds(0, w // 2)], out_hbm.at[pl.ds(r0, w // 2)],
                ssems[bb])

        def wait_write(bb, w=_TW):
            pltpu.make_async_copy(
                bouts[bb].at[pl.ds(0, w // 2)], out_hbm.at[pl.ds(0, w // 2)],
                ssems[bb]).wait()

        def transpose(bb, w=_TW):
            def tbody(u, carry):
                for par in range(2):
                    t = u * 2 + par
                    col = jnp.full((_LANES,), t, dtype=jnp.int32)
                    for c in range(cvecs):
                        v = plsc.load_gather(slabs[bb], [iotas[c], col])
                        bouts[bb][u, pl.ds(par * d_model + c * _LANES, _LANES)] = v
                return carry

            lax.fori_loop(0, w // 2, tbody, 0)

        for k in range(2):
            fire_read(base + k * _TW, k)
        for k in range(2):
            wait_read(k)
            transpose(k)
            fire_write(base + k * _TW, k)
            fire_read(base + (k + 2) * _TW, k)

        def body(o, carry):
            for bb in range(2):
                k = 2 + o * 2 + bb
                wait_read(bb)
                wait_write(bb)
                transpose(bb)
                fire_write(base + k * _TW, bb)
                fire_read(base + (k + 2) * _TW, bb)
            return carry

        lax.fori_loop(0, (blocks - 4) // 2, body, 0)

        for k in range(blocks - 2, blocks):
            bb = k % 2
            wait_read(bb)
            wait_write(bb)
            transpose(bb)
            fire_write(base + k * _TW, bb)
        for bb in range(2):
            wait_write(bb)

        if extra:
            @pl.when(wid < extra)
            def _():
                t0 = (blocks * nw + wid) * _TW
                fire_read(t0, 0)
                wait_read(0)
                transpose(0)
                fire_write(t0, 0)
                wait_write(0)

        if tail:
            trows = tail * d_model // (2 * d_model)

            @pl.when(wid == extra)
            def _():
                pltpu.sync_copy(tail_hbm, bouts[1].at[pl.ds(0, trows)])
                pltpu.sync_copy(bouts[1].at[pl.ds(0, trows)],
                                out_hbm.at[pl.ds(full_blocks * _TW // 2, trows)])

    return tr


@functools.lru_cache(maxsize=None)
def _make_gather(batch, seq_len, d_model, pos_rows, vocab):
    info = plsc.get_sparse_core_info()
    nc, ns = info.num_cores, info.num_subcores
    nw = nc * ns
    assert batch == nw * _BB
    cvecs = d_model // _LANES
    jblocks = batch // _BB
    dhi = d_model // 8

    mesh = plsc.VectorSubcoreMesh(core_axis_name="c", subcore_axis_name="s")

    scratch = [
        pltpu.VMEM((seq_len, _BB), jnp.int32),
        pltpu.VMEM((pos_rows, d_model), jnp.float32),
    ]
    scratch += [pltpu.VMEM((_BB, d_model), jnp.float32) for _ in range(2)]
    scratch += [pltpu.VMEM((dhi, 8 * _BB), jnp.float32) for _ in range(2)]
    scratch += [pltpu.SemaphoreType.DMA for _ in range(4)]

    @functools.partial(
        pl.kernel,
        mesh=mesh,
        out_type=jax.ShapeDtypeStruct(
            (seq_len, dhi, jblocks, 8 * _BB), jnp.float32),
        scratch_types=scratch,
        compiler_params=pltpu.CompilerParams(
            use_tc_tiling_on_sc=False, needs_layout_passes=False),
    )
    def enc(tok_hbm, tbl_hbm, pos_hbm, out_hbm, idx_v, pos_v,
            bin0, bin1, bout0, bout1, g0, g1, s0, s1):
        bins = (bin0, bin1)
        bouts = (bout0, bout1)
        gsems = (g0, g1)
        ssems = (s0, s1)

        wid = lax.axis_index("s") * nc + lax.axis_index("c")
        b0 = wid * _BB

        pltpu.sync_copy(pos_hbm, pos_v)
        pltpu.sync_copy(tok_hbm.at[:, pl.ds(b0, _BB)], idx_v)

        ihi = [(lax.iota(jnp.int32, _LANES) + c * _LANES) // 8 for c in range(cvecs)]
        ilo = [lax.rem(lax.iota(jnp.int32, _LANES) + c * _LANES, 8) * _BB
               for c in range(cvecs)]

        def fire_gather(s, bb):
            pltpu.async_copy(tbl_hbm.at[idx_v.at[s]], bins[bb], gsems[bb])

        def wait_gather(bb):
            pltpu.make_async_copy(tbl_hbm.at[idx_v.at[0]], bins[bb], gsems[bb]).wait()

        def fire_write(s, bb):
            pltpu.async_copy(bouts[bb], out_hbm.at[s, :, wid], ssems[bb])

        def wait_write(bb):
            pltpu.make_async_copy(bouts[bb], out_hbm.at[0, :, 0], ssems[bb]).wait()

        def compute(s, bb):
            pvs = [pos_v[s, pl.ds(c * _LANES, _LANES)] for c in range(cvecs)]

            def tbody(u, carry):
                for du in range(4):
                    t = u * 4 + du
                    colv = jnp.full((_LANES,), t, dtype=jnp.int32)
                    for c in range(cvecs):
                        v = bins[bb][t, pl.ds(c * _LANES, _LANES)] + pvs[c]
                        plsc.store_scatter(bouts[bb], [ihi[c], ilo[c] + colv], v)
                return carry

            lax.fori_loop(0, _BB // 4, tbody, 0)

        for s in range(2):
            fire_gather(s, s)
        for s in range(2):
            wait_gather(s)
            compute(s, s)
            fire_write(s, s)
            fire_gather(s + 2, s)

        def body(o, carry):
            for bb in range(2):
                s = 2 + o * 2 + bb
                wait_gather(bb)
                wait_write(bb)
                compute(s, bb)
                fire_write(s, bb)
                fire_gather(s + 2, bb)
            return carry

        lax.fori_loop(0, (seq_len - 4) // 2, body, 0)

        for s in range(seq_len - 2, seq_len):
            bb = s % 2
            wait_gather(bb)
            wait_write(bb)
            compute(s, bb)
            fire_write(s, bb)
        for bb in range(2):
            wait_write(bb)

    return enc


def kernel(token_ids, token_embed, pos_embed):
    b, s = token_ids.shape
    vocab, d = token_embed.shape
    pos_rows = pos_embed.shape[0]
    tblt = token_embed.T
    full = (vocab // _TW) * _TW
    tail_in = token_embed[full:].reshape((vocab - full) // 2, 2 * d)
    tbl_rows = _make_transpose(vocab, d)(tblt, tail_in)
    tbl_lin = tbl_rows.reshape(vocab, d)
    tok_t = token_ids.T.astype(jnp.int32)
    enc = _make_gather(b, s, d, pos_rows, vocab)
    out5 = enc(tok_t, tbl_lin, pos_embed)
    out5 = out5.reshape(s, d // 8, b // _BB, 8, _BB)
    out = jnp.transpose(out5, (2, 4, 0, 1, 3))
    return out.reshape(b, s, d)


# speedup vs baseline: 1.6619x; 1.6619x over previous
import functools

import jax
import jax.numpy as jnp
from jax import lax
from jax.experimental import pallas as pl
from jax.experimental.pallas import tpu as pltpu
from jax.experimental.pallas import tpu_sc as plsc

_LANES = 16
_BB = 128
_TW = 128


@functools.lru_cache(maxsize=None)
def _make_transpose(vocab, d_model):
    info = plsc.get_sparse_core_info()
    nc, ns = info.num_cores, info.num_subcores
    nw = nc * ns
    cvecs = d_model // _LANES
    full_blocks = vocab // _TW
    blocks = full_blocks // nw
    extra = full_blocks - blocks * nw
    tail = vocab - full_blocks * _TW

    mesh = plsc.VectorSubcoreMesh(core_axis_name="c", subcore_axis_name="s")

    scratch = [pltpu.VMEM((d_model, _TW), jnp.float32) for _ in range(2)]
    scratch += [pltpu.VMEM((_TW // 2, 2 * d_model), jnp.float32) for _ in range(2)]
    scratch += [pltpu.SemaphoreType.DMA for _ in range(4)]

    @functools.partial(
        pl.kernel,
        mesh=mesh,
        out_type=jax.ShapeDtypeStruct(
            ((vocab + 1) // 2, 2 * d_model), jnp.float32),
        scratch_types=scratch,
        compiler_params=pltpu.CompilerParams(
            use_tc_tiling_on_sc=True, needs_layout_passes=False),
    )
    def tr(tblt_hbm, tail_hbm, out_hbm, slab0, slab1, bout0, bout1,
           g0, g1, s0, s1):
        slabs = (slab0, slab1)
        bouts = (bout0, bout1)
        gsems = (g0, g1)
        ssems = (s0, s1)

        wid = lax.axis_index("s") * nc + lax.axis_index("c")
        base = wid * blocks * _TW

        iotas = [lax.iota(jnp.int32, _LANES) + c * _LANES for c in range(cvecs)]

        def fire_read(t0, bb, w=_TW):
            t0 = pl.multiple_of(t0, _TW)
            pltpu.async_copy(
                tblt_hbm.at[:, pl.ds(t0, w)], slabs[bb].at[:, pl.ds(0, w)],
                gsems[bb])

        def wait_read(bb, w=_TW):
            pltpu.make_async_copy(
                tblt_hbm.at[:, pl.ds(0, w)], slabs[bb].at[:, pl.ds(0, w)],
                gsems[bb]).wait()

        def fire_write(t0, bb, w=_TW):
            r0 = pl.multiple_of(t0 // 2, _TW // 2)
            pltpu.async_copy(
                bouts[bb].at[pl.ds(0, w // 2)], out_hbm.at[pl.ds(r0, w // 2)],
                ssems[bb])

        def wait_write(bb, w=_TW):
            pltpu.make_async_copy(
                bouts[bb].at[pl.ds(0, w // 2)], out_hbm.at[pl.ds(0, w // 2)],
                ssems[bb]).wait()

        lgroups = _TW // _LANES
        rowv = [(lax.iota(jnp.int32, _LANES) + l * _LANES) // 2
                for l in range(lgroups)]
        colb = [lax.rem(lax.iota(jnp.int32, _LANES) + l * _LANES, 2) * d_model
                for l in range(lgroups)]

        def transpose(bb, w=_TW):
            @plsc.parallel_loop(0, d_model, unroll=4)
            def _(d):
                for l in range(w // _LANES):
                    v = slabs[bb][d, pl.ds(l * _LANES, _LANES)]
                    plsc.store_scatter(bouts[bb], [rowv[l], colb[l] + d], v)

        for k in range(2):
            fire_read(base + k * _TW, k)
        for k in range(2):
            wait_read(k)
            transpose(k)
            fire_write(base + k * _TW, k)
            fire_read(base + (k + 2) * _TW, k)

        def body(o, carry):
            for bb in range(2):
                k = 2 + o * 2 + bb
                wait_read(bb)
                wait_write(bb)
                transpose(bb)
                fire_write(base + k * _TW, bb)
                fire_read(base + (k + 2) * _TW, bb)
            return carry

        lax.fori_loop(0, (blocks - 4) // 2, body, 0)

        for k in range(blocks - 2, blocks):
            bb = k % 2
            wait_read(bb)
            wait_write(bb)
            transpose(bb)
            fire_write(base + k * _TW, bb)
        for bb in range(2):
            wait_write(bb)

        if extra:
            @pl.when(wid < extra)
            def _():
                t0 = (blocks * nw + wid) * _TW
                fire_read(t0, 0)
                wait_read(0)
                transpose(0)
                fire_write(t0, 0)
                wait_write(0)

        if tail:
            trows = tail * d_model // (2 * d_model)

            @pl.when(wid == extra)
            def _():
                pltpu.sync_copy(tail_hbm, bouts[1].at[pl.ds(0, trows)])
                pltpu.sync_copy(bouts[1].at[pl.ds(0, trows)],
                                out_hbm.at[pl.ds(full_blocks * _TW // 2, trows)])

    return tr


@functools.lru_cache(maxsize=None)
def _make_gather(batch, seq_len, d_model, pos_rows, vocab):
    info = plsc.get_sparse_core_info()
    nc, ns = info.num_cores, info.num_subcores
    nw = nc * ns
    assert batch == nw * _BB
    cvecs = d_model // _LANES
    jblocks = batch // _BB
    dhi = d_model // 8

    mesh = plsc.VectorSubcoreMesh(core_axis_name="c", subcore_axis_name="s")

    scratch = [
        pltpu.VMEM((seq_len, _BB), jnp.int32),
        pltpu.VMEM((pos_rows, d_model), jnp.float32),
    ]
    scratch += [pltpu.VMEM((_BB, d_model), jnp.float32) for _ in range(2)]
    scratch += [pltpu.VMEM((dhi, 8 * _BB), jnp.float32) for _ in range(2)]
    scratch += [pltpu.SemaphoreType.DMA for _ in range(4)]

    @functools.partial(
        pl.kernel,
        mesh=mesh,
        out_type=jax.ShapeDtypeStruct(
            (seq_len, dhi, jblocks, 8 * _BB), jnp.float32),
        scratch_types=scratch,
        compiler_params=pltpu.CompilerParams(
            use_tc_tiling_on_sc=False, needs_layout_passes=False),
    )
    def enc(tok_hbm, tbl_hbm, pos_hbm, out_hbm, idx_v, pos_v,
            bin0, bin1, bout0, bout1, g0, g1, s0, s1):
        bins = (bin0, bin1)
        bouts = (bout0, bout1)
        gsems = (g0, g1)
        ssems = (s0, s1)

        wid = lax.axis_index("s") * nc + lax.axis_index("c")
        b0 = wid * _BB

        pltpu.sync_copy(pos_hbm, pos_v)
        pltpu.sync_copy(tok_hbm.at[:, pl.ds(b0, _BB)], idx_v)

        ihi = [(lax.iota(jnp.int32, _LANES) + c * _LANES) // 8 for c in range(cvecs)]
        ilo = [lax.rem(lax.iota(jnp.int32, _LANES) + c * _LANES, 8) * _BB
               for c in range(cvecs)]

        def fire_gather(s, bb):
            pltpu.async_copy(tbl_hbm.at[idx_v.at[s]], bins[bb], gsems[bb])

        def wait_gather(bb):
            pltpu.make_async_copy(tbl_hbm.at[idx_v.at[0]], bins[bb], gsems[bb]).wait()

        def fire_write(s, bb):
            pltpu.async_copy(bouts[bb], out_hbm.at[s, :, wid], ssems[bb])

        def wait_write(bb):
            pltpu.make_async_copy(bouts[bb], out_hbm.at[0, :, 0], ssems[bb]).wait()

        def compute(s, bb):
            pvs = [pos_v[s, pl.ds(c * _LANES, _LANES)] for c in range(cvecs)]

            @plsc.parallel_loop(0, _BB, unroll=8)
            def _(t):
                colv = jnp.full((_LANES,), t, dtype=jnp.int32)
                for c in range(cvecs):
                    v = bins[bb][t, pl.ds(c * _LANES, _LANES)] + pvs[c]
                    plsc.store_scatter(bouts[bb], [ihi[c], ilo[c] + colv], v)

        for s in range(2):
            fire_gather(s, s)
        for s in range(2):
            wait_gather(s)
            compute(s, s)
            fire_write(s, s)
            fire_gather(s + 2, s)

        def body(o, carry):
            for bb in range(2):
                s = 2 + o * 2 + bb
                wait_gather(bb)
                wait_write(bb)
                compute(s, bb)
                fire_write(s, bb)
                fire_gather(s + 2, bb)
            return carry

        lax.fori_loop(0, (seq_len - 4) // 2, body, 0)

        for s in range(seq_len - 2, seq_len):
            bb = s % 2
            wait_gather(bb)
            wait_write(bb)
            compute(s, bb)
            fire_write(s, bb)
        for bb in range(2):
            wait_write(bb)

    return enc


def kernel(token_ids, token_embed, pos_embed):
    b, s = token_ids.shape
    vocab, d = token_embed.shape
    pos_rows = pos_embed.shape[0]
    tblt = token_embed.T
    full = (vocab // _TW) * _TW
    tail_in = token_embed[full:].reshape((vocab - full) // 2, 2 * d)
    tbl_rows = _make_transpose(vocab, d)(tblt, tail_in)
    tbl_lin = tbl_rows.reshape(vocab, d)
    tok_t = token_ids.T.astype(jnp.int32)
    enc = _make_gather(b, s, d, pos_rows, vocab)
    out5 = enc(tok_t, tbl_lin, pos_embed)
    out5 = out5.reshape(s, d // 8, b // _BB, 8, _BB)
    out = jnp.transpose(out5, (2, 4, 0, 1, 3))
    return out.reshape(b, s, d)
